# baseline (speedup 1.0000x reference)
import functools
import math

import jax
import jax.numpy as jnp
from jax import lax
from jax.experimental import pallas as pl
from jax.experimental.pallas import tpu as pltpu
from jax.experimental.pallas import tpu_sc as plsc

N_USERS = 10000
N_ENTITIES = 40000
N_NODES = 50000
CHANNEL = 128
N_HOPS = 2
N_HEADS = 2
D_K = CHANNEL // N_HEADS
E_KG = 500000
E_EXTRA = 500000
NNZ = 200000

NC = 2
NS = 16
NW = NC * NS
L = 16

W_COLS = 144
BATCH = 64
BIG = 1 << 29

_i32 = jnp.int32
_f32 = jnp.float32


def _pad_plan(E, stage_max=2048):
    per_tile = -(-E // NS)
    per_tile = -(-per_tile // L) * L
    n_stages = -(-per_tile // stage_max)
    while (per_tile % n_stages) or ((per_tile // n_stages) % L):
        per_tile += L
        n_stages = -(-per_tile // stage_max)
    return per_tile * NS, per_tile, n_stages, per_tile // n_stages


def _zero_plan(rpt):
    assert rpt % 8 == 0
    for zr in range(min(BATCH, rpt), 0, -8):
        if zr % 8 == 0 and rpt % zr == 0:
            return rpt // zr, zr
    return rpt // 8, 8


def _iota16():
    return lax.iota(_i32, L)


ABATCH = 128


def _make_attn(E_pad):
    per_tile = E_pad // NW
    n_batches = per_tile // ABATCH
    assert n_batches * ABATCH == per_tile
    n_pairs, tail = divmod(n_batches, 2)

    mesh = plsc.VectorSubcoreMesh(core_axis_name="c", subcore_axis_name="s")

    def body(q_hbm, k_hbm, head_hbm, tail_hbm, out_hbm, *scr):
        (hidx0, tidx0, q0, k0, o0, hidx1, tidx1, q1, k1, o1,
         sq0, sk0, sq1, sk1) = scr
        w = lax.axis_index("c") * NS + lax.axis_index("s")
        base = w * per_tile
        iota = _iota16()

        def issue(b, hidx, tidx, q_buf, k_buf, sq, sk):
            bstart = base + b * ABATCH
            pltpu.sync_copy(head_hbm.at[pl.ds(bstart, ABATCH)], hidx)
            pltpu.sync_copy(tail_hbm.at[pl.ds(bstart, ABATCH)], tidx)
            dq = pltpu.async_copy(q_hbm.at[hidx], q_buf, sq)
            dk = pltpu.async_copy(k_hbm.at[tidx], k_buf, sk)
            return dq, dk

        def compute(b, q_buf, k_buf, out_stage):
            def sub_body(s16, _):
                e0 = s16 * L

                def col_body(c, acc):
                    cc = jnp.full((L,), c, _i32)
                    qc = plsc.load_gather(q_buf, [e0 + iota, cc])
                    kc = plsc.load_gather(k_buf, [e0 + iota, cc])
                    return acc + qc * kc

                acc = lax.fori_loop(0, CHANNEL, col_body,
                                    jnp.zeros((L,), _f32))
                ee = jnp.exp(acc * (1.0 / (N_HEADS * math.sqrt(D_K))))
                out_stage[pl.ds(e0, L)] = ee
                return 0

            lax.fori_loop(0, ABATCH // L, sub_body, 0)
            bstart = base + b * ABATCH
            pltpu.sync_copy(out_stage, out_hbm.at[pl.ds(bstart, ABATCH)])

        def pair_body(u, _):
            a = 2 * u
            dqa, dka = issue(a, hidx0, tidx0, q0, k0, sq0, sk0)
            dqb, dkb = issue(a + 1, hidx1, tidx1, q1, k1, sq1, sk1)
            dqa.wait()
            dka.wait()
            compute(a, q0, k0, o0)
            dqb.wait()
            dkb.wait()
            compute(a + 1, q1, k1, o1)
            return 0

        lax.fori_loop(0, n_pairs, pair_body, 0)
        if tail:
            b = n_batches - 1
            dq, dk = issue(b, hidx0, tidx0, q0, k0, sq0, sk0)
            dq.wait()
            dk.wait()
            compute(b, q0, k0, o0)

    buf_set = [
        pltpu.VMEM((ABATCH,), _i32),
        pltpu.VMEM((ABATCH,), _i32),
        pltpu.VMEM((ABATCH, CHANNEL), _f32),
        pltpu.VMEM((ABATCH, CHANNEL), _f32),
        pltpu.VMEM((ABATCH,), _f32),
    ]
    return pl.kernel(
        body,
        out_type=jax.ShapeDtypeStruct((E_pad,), _f32),
        mesh=mesh,
        compiler_params=pltpu.CompilerParams(use_tc_tiling_on_sc=False, needs_layout_passes=False),
        scratch_types=buf_set + buf_set + [pltpu.SemaphoreType.DMA] * 4,
    )


def _make_agg(E_pad, V, R, passes, use_w):
    slice_len = E_pad // NS
    assert slice_len * NS == E_pad
    n_stages = 1
    stage = slice_len
    while stage > 2048:
        n_stages *= 2
        stage = slice_len // n_stages
    assert stage * n_stages == slice_len and stage % L == 0
    cap = stage + 2 * BATCH
    rpt = R // NS
    assert rpt * NS == R
    nz, zr = _zero_plan(rpt)
    acc_rows = R + L
    out_rows = NC * passes * R

    mesh = plsc.VectorSubcoreMesh(core_axis_name="c", subcore_axis_name="s")

    def body(src_hbm, head_hbm, tail_hbm, scale_hbm, et_hbm, w_hbm, out_hbm,
             *scr):
        if use_w:
            (h_stage, t_stage, s_stage, e_stage, w_vmem,
             gidx, dstb, sclb, etb, gb0, gb1, st2a, st2b,
             g_buf0, g_buf1, s_buf0, s_buf1, sg0, sg1, ss0, ss1, acc) = scr
        else:
            (h_stage, t_stage, s_stage,
             gidx, dstb, sclb, gb0, gb1, st2a, st2b,
             g_buf0, g_buf1, s_buf0, s_buf1, sg0, sg1, ss0, ss1, acc) = scr
            e_stage = etb = w_vmem = None
        c_idx = lax.axis_index("c")
        s_idx = lax.axis_index("s")
        iota = _iota16()
        zeros_f = jnp.zeros((L,), _f32)
        zeros_i = jnp.zeros((L,), _i32)

        if use_w:
            pltpu.sync_copy(w_hbm, w_vmem)

        for p in range(passes):
            chunk = c_idx * passes + p
            lo = chunk * R

            def zrow(r, _):
                for c9 in range(W_COLS // L):
                    s_buf0[r, pl.ds(c9 * L, L)] = zeros_f
                return 0
            lax.fori_loop(0, BATCH, zrow, 0)
            row0 = s_idx * rpt
            for i in range(nz):
                pltpu.sync_copy(s_buf0.at[pl.ds(0, zr)],
                                acc.at[pl.ds(row0 + i * zr, zr)])
            plsc.subcore_barrier()

            def stage_body(o, _):
                gbase = s_idx * slice_len + o * stage
                pltpu.sync_copy(head_hbm.at[pl.ds(gbase, stage)], h_stage)
                pltpu.sync_copy(tail_hbm.at[pl.ds(gbase, stage)], t_stage)
                pltpu.sync_copy(scale_hbm.at[pl.ds(gbase, stage)], s_stage)
                if use_w:
                    pltpu.sync_copy(et_hbm.at[pl.ds(gbase, stage)], e_stage)

                def blk_body(j, cnt):
                    off = j * L
                    h16 = h_stage[pl.ds(off, L)]
                    m = (h16 >= lo) & (h16 < lo + R)
                    t16 = t_stage[pl.ds(off, L)]
                    s16 = s_stage[pl.ds(off, L)]
                    mi = plsc.cumsum(m.astype(_i32))
                    pos = cnt + mi - 1
                    plsc.store_scatter(gidx, [pos], t16, mask=m)
                    plsc.store_scatter(dstb, [pos], h16 - lo, mask=m)
                    plsc.store_scatter(sclb, [pos], s16, mask=m)
                    if use_w:
                        e16 = e_stage[pl.ds(off, L)]
                        plsc.store_scatter(etb, [pos], e16, mask=m)
                    return cnt + jnp.max(mi)

                cnt = lax.fori_loop(0, stage // L, blk_body, jnp.int32(0))

                for k in range(2 * BATCH // L):
                    base = cnt + k * L
                    gidx[pl.ds(base, L)] = zeros_i
                    dstb[pl.ds(base, L)] = jnp.full((L,), R, _i32)
                    sclb[pl.ds(base, L)] = zeros_f
                    if use_w:
                        etb[pl.ds(base, L)] = zeros_i

                def fill_issue(boff, gb, st2, g_buf, sg):
                    for k8 in range(BATCH // L):
                        gb[pl.ds(k8 * L, L)] = gidx[pl.ds(boff + k8 * L, L)]
                        st2[0, pl.ds(k8 * L, L)] = dstb[pl.ds(boff + k8 * L, L)]
                    return pltpu.async_copy(src_hbm.at[gb], g_buf, sg)

                def scale_batch(boff, g_buf, s_buf):
                    def ebody(e, _):
                        eidx = jnp.full((L,), boff + e, _i32)
                        sb = plsc.load_gather(sclb, [eidx])
                        if use_w:
                            et_b = plsc.load_gather(etb, [eidx])
                        for c8 in range(CHANNEL // L):
                            v = g_buf[e, pl.ds(c8 * L, L)]
                            if use_w:
                                wv = plsc.load_gather(
                                    w_vmem, [et_b * CHANNEL + c8 * L + iota])
                                v = v * wv
                            s_buf[e, pl.ds(c8 * L, L)] = v * sb
                        extras = jnp.where(
                            iota == 0, sb,
                            jnp.where(iota == 1, jnp.ones((L,), _f32),
                                      zeros_f))
                        s_buf[e, pl.ds(CHANNEL, L)] = extras
                        return 0

                    lax.fori_loop(0, BATCH, ebody, 0)

                npairs = (cnt + (2 * BATCH - 1)) // (2 * BATCH)

                def pair_body(u, _):
                    boff_a = u * 2 * BATCH
                    boff_b = boff_a + BATCH
                    da = fill_issue(boff_a, gb0, st2a, g_buf0, sg0)
                    db = fill_issue(boff_b, gb1, st2b, g_buf1, sg1)
                    da.wait()
                    scale_batch(boff_a, g_buf0, s_buf0)
                    dsa = pltpu.async_copy(s_buf0, acc.at[st2a.at[0]], ss0,
                                           add=True)
                    db.wait()
                    scale_batch(boff_b, g_buf1, s_buf1)
                    dsb = pltpu.async_copy(s_buf1, acc.at[st2b.at[0]], ss1,
                                           add=True)
                    dsa.wait()
                    dsb.wait()
                    return 0

                lax.fori_loop(0, npairs, pair_body, 0)
                return 0

            lax.fori_loop(0, n_stages, stage_body, 0)
            plsc.subcore_barrier()

            out_row0 = chunk * R + row0
            pltpu.sync_copy(acc.at[pl.ds(row0, rpt)],
                            out_hbm.at[pl.ds(out_row0, rpt)])
            plsc.subcore_barrier()

    return pl.kernel(
        body,
        out_type=jax.ShapeDtypeStruct((out_rows, W_COLS), _f32),
        mesh=mesh,
        compiler_params=pltpu.CompilerParams(use_tc_tiling_on_sc=False, needs_layout_passes=False),
        scratch_types=(
            [pltpu.VMEM((stage,), _i32),
             pltpu.VMEM((stage,), _i32),
             pltpu.VMEM((stage,), _f32)]
            + ([pltpu.VMEM((stage,), _i32),
                pltpu.VMEM((10 * CHANNEL,), _f32)] if use_w else [])
            + [pltpu.VMEM((cap,), _i32),
               pltpu.VMEM((cap,), _i32)]
            + [pltpu.VMEM((cap,), _f32)]
            + ([pltpu.VMEM((cap,), _i32)] if use_w else [])
            + [pltpu.VMEM((BATCH,), _i32),
               pltpu.VMEM((BATCH,), _i32),
               pltpu.VMEM((1, BATCH), _i32),
               pltpu.VMEM((1, BATCH), _i32),
               pltpu.VMEM((BATCH, CHANNEL), _f32),
               pltpu.VMEM((BATCH, CHANNEL), _f32),
               pltpu.VMEM((BATCH, W_COLS), _f32),
               pltpu.VMEM((BATCH, W_COLS), _f32),
               pltpu.SemaphoreType.DMA,
               pltpu.SemaphoreType.DMA,
               pltpu.SemaphoreType.DMA,
               pltpu.SemaphoreType.DMA,
               pltpu.VMEM_SHARED((acc_rows, W_COLS), _f32)]
        ),
    )


_BM = 1000


def _qk_body(x_ref, wq_ref, wk_ref, q_ref, k_ref):
    x = x_ref[...]
    q_ref[...] = jnp.dot(x, wq_ref[...], preferred_element_type=_f32)
    k_ref[...] = jnp.dot(x, wk_ref[...], preferred_element_type=_f32)


def _qk_project(x, wq, wk):
    n = x.shape[0]
    grid = n // _BM
    return pl.pallas_call(
        _qk_body,
        grid=(grid,),
        in_specs=[
            pl.BlockSpec((_BM, CHANNEL), lambda i: (i, 0)),
            pl.BlockSpec((CHANNEL, CHANNEL), lambda i: (0, 0)),
            pl.BlockSpec((CHANNEL, CHANNEL), lambda i: (0, 0)),
        ],
        out_specs=[
            pl.BlockSpec((_BM, CHANNEL), lambda i: (i, 0)),
            pl.BlockSpec((_BM, CHANNEL), lambda i: (i, 0)),
        ],
        out_shape=[
            jax.ShapeDtypeStruct((n, CHANNEL), _f32),
            jax.ShapeDtypeStruct((n, CHANNEL), _f32),
        ],
    )(x, wq, wk)


def _l2n(x):
    n = jnp.sqrt(jnp.sum(x * x, axis=1, keepdims=True))
    return x / jnp.maximum(n, 1e-12)


def _fin_ent_body(raw_ref, res_ref, mean_ref, n_ref, newres_ref):
    raw = raw_ref[...]
    s = raw[:, :CHANNEL]
    c = raw[:, CHANNEL + 1]
    mean = s / jnp.maximum(c, 1.0)[:, None]
    nrm = _l2n(mean)
    mean_ref[...] = mean
    n_ref[...] = nrm
    newres_ref[...] = res_ref[...] + nrm


def _fin_node_body(raw_ref, res_ref, n_ref, newres_ref):
    raw = raw_ref[...]
    s = raw[:, :CHANNEL]
    z = raw[:, CHANNEL]
    c = raw[:, CHANNEL + 1]
    mean = s / (jnp.maximum(z, 1e-30) * jnp.maximum(c, 1.0))[:, None]
    nrm = _l2n(mean)
    n_ref[...] = nrm
    newres_ref[...] = res_ref[...] + nrm


def _fin_user_body(raw_ref, res_ref, newres_ref):
    raw = raw_ref[...]
    nrm = _l2n(raw[:, :CHANNEL])
    newres_ref[...] = res_ref[...] + nrm


def _finalize(body, raw, res, n_out):
    n = res.shape[0]
    grid = n // _BM
    outs = [jax.ShapeDtypeStruct((n, CHANNEL), _f32)] * n_out
    return pl.pallas_call(
        body,
        grid=(grid,),
        in_specs=[
            pl.BlockSpec((_BM, W_COLS), lambda i: (i, 0)),
            pl.BlockSpec((_BM, CHANNEL), lambda i: (i, 0)),
        ],
        out_specs=[pl.BlockSpec((_BM, CHANNEL), lambda i: (i, 0))] * n_out,
        out_shape=outs,
    )(raw, res)


_EPAD_ATTN, _, _, _ = _pad_plan(E_EXTRA, 2048)
_EPAD_KG = _pad_plan(E_KG)[0]
_EPAD_EX = _pad_plan(E_EXTRA)[0]
_NNZ_PAD = _pad_plan(NNZ)[0]

_BISECT = 0

_attn = _make_attn(_EPAD_ATTN)
_agg_ent = _make_agg(_EPAD_KG, N_ENTITIES, 8576, 3, True)
_agg_node = _make_agg(_EPAD_EX, N_NODES, 8576, 3, False)
_agg_user = _make_agg(_NNZ_PAD, N_ENTITIES, 5120, 1, False)


def _pad_i(x, n, val):
    return jnp.concatenate([x, jnp.full((n - x.shape[0],), val, x.dtype)])


def kernel(user_emb, entity_emb, edge_index, edge_type, extra_edge_index,
           extra_edge_type, interact_rows, interact_cols, interact_vals,
           weight, extra_weight, W_Q, W_K):
    del extra_edge_type, extra_weight
    node0 = jnp.concatenate([user_emb, entity_emb], axis=0)

    q, k = _qk_project(node0, W_Q, W_K)
    head_ex = extra_edge_index[0]
    tail_ex = extra_edge_index[1]
    head_ex0 = _pad_i(head_ex, _EPAD_ATTN, 0)
    tail_ex0 = _pad_i(tail_ex, _EPAD_ATTN, 0)
    ee = _attn(q, k, head_ex0, tail_ex0)
    if _BISECT == 1:
        return (entity_emb + ee[0], user_emb, node0)
    if _BISECT == 2:
        head_kg = _pad_i(edge_index[0], _EPAD_KG, BIG)
        tail_kg = _pad_i(edge_index[1], _EPAD_KG, 0)
        et_kg = _pad_i(edge_type - 1, _EPAD_KG, 0)
        ones_kg = jnp.ones((_EPAD_KG,), _f32)
        raw_ent = _agg_ent(entity_emb, head_kg, tail_kg, ones_kg, et_kg,
                           weight.reshape(-1))
        return (entity_emb + raw_ent[0, 0] + ee[0], user_emb, node0)

    head_kg = _pad_i(edge_index[0], _EPAD_KG, BIG)
    tail_kg = _pad_i(edge_index[1], _EPAD_KG, 0)
    et_kg = _pad_i(edge_type - 1, _EPAD_KG, 0)
    ones_kg = jnp.ones((_EPAD_KG,), _f32)
    w_flat = weight.reshape(-1)

    head_exb = _pad_i(head_ex, _EPAD_EX, BIG)
    zeros_i = jnp.zeros((_EPAD_EX,), _i32)
    dummy_w = jnp.zeros((10 * CHANNEL,), _f32)

    rows_p = _pad_i(interact_rows, _NNZ_PAD, BIG)
    cols_p = _pad_i(interact_cols, _NNZ_PAD, 0)
    vals_p = _pad_i(interact_vals, _NNZ_PAD, 0.0)
    zeros_nnz = jnp.zeros((_NNZ_PAD,), _i32)

    ent_res, node_res, user_res = entity_emb, node0, user_emb
    x_ent, x_node = entity_emb, node0
    for _ in range(N_HOPS):
        raw_ent = _agg_ent(x_ent, head_kg, tail_kg, ones_kg, et_kg, w_flat)
        ent_mean, ent_n, ent_res = _finalize(
            _fin_ent_body, raw_ent[:N_ENTITIES], ent_res, 3)

        raw_node = _agg_node(x_node, head_exb, tail_ex0[: _EPAD_EX], ee,
                             zeros_i, dummy_w)
        node_n, node_res = _finalize(
            _fin_node_body, raw_node[:N_NODES], node_res, 2)

        raw_user = _agg_user(ent_mean, rows_p, cols_p, vals_p, zeros_nnz,
                             dummy_w)
        (user_res,) = _finalize(
            _fin_user_body, raw_user[:N_USERS], user_res, 1)

        x_ent, x_node = ent_n, node_n

    return (ent_res, user_res, node_res)


# speedup vs baseline: 1.6944x; 1.6944x over previous
import functools
import math

import jax
import jax.numpy as jnp
from jax import lax
from jax.experimental import pallas as pl
from jax.experimental.pallas import tpu as pltpu
from jax.experimental.pallas import tpu_sc as plsc

N_USERS = 10000
N_ENTITIES = 40000
N_NODES = 50000
CHANNEL = 128
N_HOPS = 2
N_HEADS = 2
D_K = CHANNEL // N_HEADS
E_KG = 500000
E_EXTRA = 500000
NNZ = 200000

NC = 2
NS = 16
NW = NC * NS
L = 16

W_COLS = 144
BATCH = 64
BIG = 1 << 29

_i32 = jnp.int32
_f32 = jnp.float32


def _pad_plan(E, stage_max=2048):
    per_tile = -(-E // NS)
    per_tile = -(-per_tile // L) * L
    n_stages = -(-per_tile // stage_max)
    while (per_tile % n_stages) or ((per_tile // n_stages) % L):
        per_tile += L
        n_stages = -(-per_tile // stage_max)
    return per_tile * NS, per_tile, n_stages, per_tile // n_stages


def _zero_plan(rpt):
    assert rpt % 8 == 0
    for zr in range(min(BATCH, rpt), 0, -8):
        if zr % 8 == 0 and rpt % zr == 0:
            return rpt // zr, zr
    return rpt // 8, 8


def _iota16():
    return lax.iota(_i32, L)


ABATCH = 128


def _make_attn(E_pad):
    per_tile = E_pad // NW
    n_batches = per_tile // ABATCH
    assert n_batches * ABATCH == per_tile
    n_pairs, tail = divmod(n_batches, 2)

    mesh = plsc.VectorSubcoreMesh(core_axis_name="c", subcore_axis_name="s")

    def body(q_hbm, k_hbm, head_hbm, tail_hbm, out_hbm, *scr):
        (hidx0, tidx0, q0, k0, o0, hidx1, tidx1, q1, k1, o1,
         sq0, sk0, sq1, sk1) = scr
        w = lax.axis_index("c") * NS + lax.axis_index("s")
        base = w * per_tile
        iota = _iota16()

        def issue(b, hidx, tidx, q_buf, k_buf, sq, sk):
            bstart = base + b * ABATCH
            pltpu.sync_copy(head_hbm.at[pl.ds(bstart, ABATCH)], hidx)
            pltpu.sync_copy(tail_hbm.at[pl.ds(bstart, ABATCH)], tidx)
            dq = pltpu.async_copy(q_hbm.at[hidx], q_buf, sq)
            dk = pltpu.async_copy(k_hbm.at[tidx], k_buf, sk)
            return dq, dk

        def compute(b, q_buf, k_buf, out_stage):
            def sub_body(s16, _):
                e0 = s16 * L

                def col_body(c, acc):
                    cc = jnp.full((L,), c, _i32)
                    qc = plsc.load_gather(q_buf, [e0 + iota, cc])
                    kc = plsc.load_gather(k_buf, [e0 + iota, cc])
                    return acc + qc * kc

                acc = lax.fori_loop(0, CHANNEL, col_body,
                                    jnp.zeros((L,), _f32))
                ee = jnp.exp(acc * (1.0 / (N_HEADS * math.sqrt(D_K))))
                out_stage[pl.ds(e0, L)] = ee
                return 0

            lax.fori_loop(0, ABATCH // L, sub_body, 0)
            bstart = base + b * ABATCH
            pltpu.sync_copy(out_stage, out_hbm.at[pl.ds(bstart, ABATCH)])

        def pair_body(u, _):
            a = 2 * u
            dqa, dka = issue(a, hidx0, tidx0, q0, k0, sq0, sk0)
            dqb, dkb = issue(a + 1, hidx1, tidx1, q1, k1, sq1, sk1)
            dqa.wait()
            dka.wait()
            compute(a, q0, k0, o0)
            dqb.wait()
            dkb.wait()
            compute(a + 1, q1, k1, o1)
            return 0

        lax.fori_loop(0, n_pairs, pair_body, 0)
        if tail:
            b = n_batches - 1
            dq, dk = issue(b, hidx0, tidx0, q0, k0, sq0, sk0)
            dq.wait()
            dk.wait()
            compute(b, q0, k0, o0)

    buf_set = [
        pltpu.VMEM((ABATCH,), _i32),
        pltpu.VMEM((ABATCH,), _i32),
        pltpu.VMEM((ABATCH, CHANNEL), _f32),
        pltpu.VMEM((ABATCH, CHANNEL), _f32),
        pltpu.VMEM((ABATCH,), _f32),
    ]
    return pl.kernel(
        body,
        out_type=jax.ShapeDtypeStruct((E_pad,), _f32),
        mesh=mesh,
        compiler_params=pltpu.CompilerParams(use_tc_tiling_on_sc=False, needs_layout_passes=False),
        scratch_types=buf_set + buf_set + [pltpu.SemaphoreType.DMA] * 4,
    )


def _make_agg(E_pad, V, R, passes, use_w):
    slice_len = E_pad // NS
    assert slice_len * NS == E_pad
    n_stages = 1
    stage = slice_len
    while stage > 2048:
        n_stages *= 2
        stage = slice_len // n_stages
    assert stage * n_stages == slice_len and stage % L == 0
    cap = stage + BATCH
    rpt = R // NS
    assert rpt * NS == R
    nz, zr = _zero_plan(rpt)
    acc_rows = R + L
    out_rows = NC * passes * R

    mesh = plsc.VectorSubcoreMesh(core_axis_name="c", subcore_axis_name="s")

    def body(src_hbm, head_hbm, tail_hbm, scale_hbm, et_hbm, w_hbm, out_hbm,
             *scr):
        if use_w:
            (h_stage, t_stage, s_stage, e_stage, w_vmem,
             gidx, dstb, sclb, etb, gb_idx, stage2, g_buf, s_buf, acc) = scr
        else:
            (h_stage, t_stage, s_stage,
             gidx, dstb, sclb, gb_idx, stage2, g_buf, s_buf, acc) = scr
            e_stage = etb = w_vmem = None
        c_idx = lax.axis_index("c")
        s_idx = lax.axis_index("s")
        iota = _iota16()
        zeros_f = jnp.zeros((L,), _f32)
        zeros_i = jnp.zeros((L,), _i32)

        if use_w:
            pltpu.sync_copy(w_hbm, w_vmem)

        for p in range(passes):
            chunk = c_idx * passes + p
            lo = chunk * R

            def zrow(r, _):
                for c9 in range(W_COLS // L):
                    s_buf[r, pl.ds(c9 * L, L)] = zeros_f
                return 0
            lax.fori_loop(0, BATCH, zrow, 0)
            row0 = s_idx * rpt
            for i in range(nz):
                pltpu.sync_copy(s_buf.at[pl.ds(0, zr)],
                                acc.at[pl.ds(row0 + i * zr, zr)])
            plsc.subcore_barrier()

            def stage_body(o, _):
                gbase = s_idx * slice_len + o * stage
                pltpu.sync_copy(head_hbm.at[pl.ds(gbase, stage)], h_stage)
                pltpu.sync_copy(tail_hbm.at[pl.ds(gbase, stage)], t_stage)
                pltpu.sync_copy(scale_hbm.at[pl.ds(gbase, stage)], s_stage)
                if use_w:
                    pltpu.sync_copy(et_hbm.at[pl.ds(gbase, stage)], e_stage)

                def blk_body(j, cnt):
                    off = j * L
                    h16 = h_stage[pl.ds(off, L)]
                    m = (h16 >= lo) & (h16 < lo + R)
                    t16 = t_stage[pl.ds(off, L)]
                    s16 = s_stage[pl.ds(off, L)]
                    mi = plsc.cumsum(m.astype(_i32))
                    pos = cnt + mi - 1
                    plsc.store_scatter(gidx, [pos], t16, mask=m)
                    plsc.store_scatter(dstb, [pos], h16 - lo, mask=m)
                    plsc.store_scatter(sclb, [pos], s16, mask=m)
                    if use_w:
                        e16 = e_stage[pl.ds(off, L)]
                        plsc.store_scatter(etb, [pos], e16, mask=m)
                    return cnt + jnp.max(mi)

                cnt = lax.fori_loop(0, stage // L, blk_body, jnp.int32(0))

                for k in range(BATCH // L):
                    base = cnt + k * L
                    gidx[pl.ds(base, L)] = zeros_i
                    dstb[pl.ds(base, L)] = jnp.full((L,), R, _i32)
                    sclb[pl.ds(base, L)] = zeros_f
                    if use_w:
                        etb[pl.ds(base, L)] = zeros_i

                nb = (cnt + (BATCH - 1)) // BATCH

                def batch_body(b, _):
                    boff = b * BATCH
                    for k8 in range(BATCH // L):
                        gb_idx[pl.ds(k8 * L, L)] = gidx[pl.ds(boff + k8 * L, L)]
                        stage2[0, pl.ds(k8 * L, L)] = dstb[pl.ds(boff + k8 * L, L)]
                    pltpu.sync_copy(src_hbm.at[gb_idx], g_buf)

                    def ebody(e, _):
                        eidx = jnp.full((L,), boff + e, _i32)
                        sb = plsc.load_gather(sclb, [eidx])
                        if use_w:
                            et_b = plsc.load_gather(etb, [eidx])
                        for c8 in range(CHANNEL // L):
                            v = g_buf[e, pl.ds(c8 * L, L)]
                            if use_w:
                                wv = plsc.load_gather(
                                    w_vmem, [et_b * CHANNEL + c8 * L + iota])
                                v = v * wv
                            s_buf[e, pl.ds(c8 * L, L)] = v * sb
                        extras = jnp.where(
                            iota == 0, sb,
                            jnp.where(iota == 1, jnp.ones((L,), _f32),
                                      zeros_f))
                        s_buf[e, pl.ds(CHANNEL, L)] = extras
                        return 0

                    lax.fori_loop(0, BATCH, ebody, 0)
                    pltpu.sync_copy(s_buf, acc.at[stage2.at[0]], add=True)
                    return 0

                lax.fori_loop(0, nb, batch_body, 0)
                return 0

            lax.fori_loop(0, n_stages, stage_body, 0)
            plsc.subcore_barrier()

            out_row0 = chunk * R + row0
            pltpu.sync_copy(acc.at[pl.ds(row0, rpt)],
                            out_hbm.at[pl.ds(out_row0, rpt)])
            plsc.subcore_barrier()

    return pl.kernel(
        body,
        out_type=jax.ShapeDtypeStruct((out_rows, W_COLS), _f32),
        mesh=mesh,
        compiler_params=pltpu.CompilerParams(use_tc_tiling_on_sc=False, needs_layout_passes=False),
        scratch_types=(
            [pltpu.VMEM((stage,), _i32),
             pltpu.VMEM((stage,), _i32),
             pltpu.VMEM((stage,), _f32)]
            + ([pltpu.VMEM((stage,), _i32),
                pltpu.VMEM((10 * CHANNEL,), _f32)] if use_w else [])
            + [pltpu.VMEM((cap,), _i32),
               pltpu.VMEM((cap,), _i32)]
            + [pltpu.VMEM((cap,), _f32)]
            + ([pltpu.VMEM((cap,), _i32)] if use_w else [])
            + [pltpu.VMEM((BATCH,), _i32),
               pltpu.VMEM((1, BATCH), _i32),
               pltpu.VMEM((BATCH, CHANNEL), _f32),
               pltpu.VMEM((BATCH, W_COLS), _f32),
               pltpu.VMEM_SHARED((acc_rows, W_COLS), _f32)]
        ),
    )


_BM = 1000


def _qk_body(x_ref, wq_ref, wk_ref, q_ref, k_ref):
    x = x_ref[...]
    q_ref[...] = jnp.dot(x, wq_ref[...], preferred_element_type=_f32)
    k_ref[...] = jnp.dot(x, wk_ref[...], preferred_element_type=_f32)


def _qk_project(x, wq, wk):
    n = x.shape[0]
    grid = n // _BM
    return pl.pallas_call(
        _qk_body,
        grid=(grid,),
        in_specs=[
            pl.BlockSpec((_BM, CHANNEL), lambda i: (i, 0)),
            pl.BlockSpec((CHANNEL, CHANNEL), lambda i: (0, 0)),
            pl.BlockSpec((CHANNEL, CHANNEL), lambda i: (0, 0)),
        ],
        out_specs=[
            pl.BlockSpec((_BM, CHANNEL), lambda i: (i, 0)),
            pl.BlockSpec((_BM, CHANNEL), lambda i: (i, 0)),
        ],
        out_shape=[
            jax.ShapeDtypeStruct((n, CHANNEL), _f32),
            jax.ShapeDtypeStruct((n, CHANNEL), _f32),
        ],
    )(x, wq, wk)


def _l2n(x):
    n = jnp.sqrt(jnp.sum(x * x, axis=1, keepdims=True))
    return x / jnp.maximum(n, 1e-12)


def _fin_ent_body(raw_ref, res_ref, mean_ref, n_ref, newres_ref):
    raw = raw_ref[...]
    s = raw[:, :CHANNEL]
    c = raw[:, CHANNEL + 1]
    mean = s / jnp.maximum(c, 1.0)[:, None]
    nrm = _l2n(mean)
    mean_ref[...] = mean
    n_ref[...] = nrm
    newres_ref[...] = res_ref[...] + nrm


def _fin_node_body(raw_ref, res_ref, n_ref, newres_ref):
    raw = raw_ref[...]
    s = raw[:, :CHANNEL]
    z = raw[:, CHANNEL]
    c = raw[:, CHANNEL + 1]
    mean = s / (jnp.maximum(z, 1e-30) * jnp.maximum(c, 1.0))[:, None]
    nrm = _l2n(mean)
    n_ref[...] = nrm
    newres_ref[...] = res_ref[...] + nrm


def _fin_user_body(raw_ref, res_ref, newres_ref):
    raw = raw_ref[...]
    nrm = _l2n(raw[:, :CHANNEL])
    newres_ref[...] = res_ref[...] + nrm


def _finalize(body, raw, res, n_out):
    n = res.shape[0]
    grid = n // _BM
    outs = [jax.ShapeDtypeStruct((n, CHANNEL), _f32)] * n_out
    return pl.pallas_call(
        body,
        grid=(grid,),
        in_specs=[
            pl.BlockSpec((_BM, W_COLS), lambda i: (i, 0)),
            pl.BlockSpec((_BM, CHANNEL), lambda i: (i, 0)),
        ],
        out_specs=[pl.BlockSpec((_BM, CHANNEL), lambda i: (i, 0))] * n_out,
        out_shape=outs,
    )(raw, res)


_EPAD_ATTN, _, _, _ = _pad_plan(E_EXTRA, 2048)
_EPAD_KG = _pad_plan(E_KG)[0]
_EPAD_EX = _pad_plan(E_EXTRA)[0]
_NNZ_PAD = _pad_plan(NNZ)[0]

_BISECT = 0

_attn = _make_attn(_EPAD_ATTN)
_agg_ent = _make_agg(_EPAD_KG, N_ENTITIES, 10240, 2, True)
_agg_node = _make_agg(_EPAD_EX, N_NODES, 10240, 3, False)
_agg_user = _make_agg(_NNZ_PAD, N_ENTITIES, 5120, 1, False)


def _pad_i(x, n, val):
    return jnp.concatenate([x, jnp.full((n - x.shape[0],), val, x.dtype)])


def kernel(user_emb, entity_emb, edge_index, edge_type, extra_edge_index,
           extra_edge_type, interact_rows, interact_cols, interact_vals,
           weight, extra_weight, W_Q, W_K):
    del extra_edge_type, extra_weight
    node0 = jnp.concatenate([user_emb, entity_emb], axis=0)

    q, k = _qk_project(node0, W_Q, W_K)
    head_ex = extra_edge_index[0]
    tail_ex = extra_edge_index[1]
    head_ex0 = _pad_i(head_ex, _EPAD_ATTN, 0)
    tail_ex0 = _pad_i(tail_ex, _EPAD_ATTN, 0)
    ee = _attn(q, k, head_ex0, tail_ex0)
    if _BISECT == 1:
        return (entity_emb + ee[0], user_emb, node0)
    if _BISECT == 2:
        head_kg = _pad_i(edge_index[0], _EPAD_KG, BIG)
        tail_kg = _pad_i(edge_index[1], _EPAD_KG, 0)
        et_kg = _pad_i(edge_type - 1, _EPAD_KG, 0)
        ones_kg = jnp.ones((_EPAD_KG,), _f32)
        raw_ent = _agg_ent(entity_emb, head_kg, tail_kg, ones_kg, et_kg,
                           weight.reshape(-1))
        return (entity_emb + raw_ent[0, 0] + ee[0], user_emb, node0)

    head_kg = _pad_i(edge_index[0], _EPAD_KG, BIG)
    tail_kg = _pad_i(edge_index[1], _EPAD_KG, 0)
    et_kg = _pad_i(edge_type - 1, _EPAD_KG, 0)
    ones_kg = jnp.ones((_EPAD_KG,), _f32)
    w_flat = weight.reshape(-1)

    head_exb = _pad_i(head_ex, _EPAD_EX, BIG)
    zeros_i = jnp.zeros((_EPAD_EX,), _i32)
    dummy_w = jnp.zeros((10 * CHANNEL,), _f32)

    rows_p = _pad_i(interact_rows, _NNZ_PAD, BIG)
    cols_p = _pad_i(interact_cols, _NNZ_PAD, 0)
    vals_p = _pad_i(interact_vals, _NNZ_PAD, 0.0)
    zeros_nnz = jnp.zeros((_NNZ_PAD,), _i32)

    ent_res, node_res, user_res = entity_emb, node0, user_emb
    x_ent, x_node = entity_emb, node0
    for _ in range(N_HOPS):
        raw_ent = _agg_ent(x_ent, head_kg, tail_kg, ones_kg, et_kg, w_flat)
        ent_mean, ent_n, ent_res = _finalize(
            _fin_ent_body, raw_ent[:N_ENTITIES], ent_res, 3)

        raw_node = _agg_node(x_node, head_exb, tail_ex0[: _EPAD_EX], ee,
                             zeros_i, dummy_w)
        node_n, node_res = _finalize(
            _fin_node_body, raw_node[:N_NODES], node_res, 2)

        raw_user = _agg_user(ent_mean, rows_p, cols_p, vals_p, zeros_nnz,
                             dummy_w)
        (user_res,) = _finalize(
            _fin_user_body, raw_user[:N_USERS], user_res, 1)

        x_ent, x_node = ent_n, node_n

    return (ent_res, user_res, node_res)


# speedup vs baseline: 1.6972x; 1.0016x over previous
import functools
import math

import jax
import jax.numpy as jnp
from jax import lax
from jax.experimental import pallas as pl
from jax.experimental.pallas import tpu as pltpu
from jax.experimental.pallas import tpu_sc as plsc

N_USERS = 10000
N_ENTITIES = 40000
N_NODES = 50000
CHANNEL = 128
N_HOPS = 2
N_HEADS = 2
D_K = CHANNEL // N_HEADS
E_KG = 500000
E_EXTRA = 500000
NNZ = 200000

NC = 2
NS = 16
NW = NC * NS
L = 16

W_COLS = 144
BATCH = 64
BIG = 1 << 29

_i32 = jnp.int32
_f32 = jnp.float32


def _pad_plan(E, stage_max=2048):
    per_tile = -(-E // NS)
    per_tile = -(-per_tile // L) * L
    n_stages = -(-per_tile // stage_max)
    while (per_tile % n_stages) or ((per_tile // n_stages) % L):
        per_tile += L
        n_stages = -(-per_tile // stage_max)
    return per_tile * NS, per_tile, n_stages, per_tile // n_stages


def _zero_plan(rpt):
    assert rpt % 8 == 0
    for zr in range(min(BATCH, rpt), 0, -8):
        if zr % 8 == 0 and rpt % zr == 0:
            return rpt // zr, zr
    return rpt // 8, 8


def _iota16():
    return lax.iota(_i32, L)


ABATCH = 128


def _make_attn(E_pad):
    per_tile = E_pad // NW
    n_batches = per_tile // ABATCH
    assert n_batches * ABATCH == per_tile
    n_pairs, tail = divmod(n_batches, 2)

    mesh = plsc.VectorSubcoreMesh(core_axis_name="c", subcore_axis_name="s")

    def body(q_hbm, k_hbm, head_hbm, tail_hbm, out_hbm, *scr):
        (hidx0, tidx0, q0, k0, o0, hidx1, tidx1, q1, k1, o1,
         sq0, sk0, sq1, sk1) = scr
        w = lax.axis_index("c") * NS + lax.axis_index("s")
        base = w * per_tile
        iota = _iota16()

        def issue(b, hidx, tidx, q_buf, k_buf, sq, sk):
            bstart = base + b * ABATCH
            pltpu.sync_copy(head_hbm.at[pl.ds(bstart, ABATCH)], hidx)
            pltpu.sync_copy(tail_hbm.at[pl.ds(bstart, ABATCH)], tidx)
            dq = pltpu.async_copy(q_hbm.at[hidx], q_buf, sq)
            dk = pltpu.async_copy(k_hbm.at[tidx], k_buf, sk)
            return dq, dk

        def compute(b, q_buf, k_buf, out_stage):
            def sub_body(s16, _):
                e0 = s16 * L
                erow = e0 + iota

                def col_body(c8, acc):
                    for dc in range(8):
                        cc = jnp.full((L,), c8 * 8 + dc, _i32)
                        qc = plsc.load_gather(q_buf, [erow, cc])
                        kc = plsc.load_gather(k_buf, [erow, cc])
                        acc = acc + qc * kc
                    return acc

                acc = lax.fori_loop(0, CHANNEL // 8, col_body,
                                    jnp.zeros((L,), _f32))
                ee = jnp.exp(acc * (1.0 / (N_HEADS * math.sqrt(D_K))))
                out_stage[pl.ds(e0, L)] = ee
                return 0

            lax.fori_loop(0, ABATCH // L, sub_body, 0)
            bstart = base + b * ABATCH
            pltpu.sync_copy(out_stage, out_hbm.at[pl.ds(bstart, ABATCH)])

        def pair_body(u, _):
            a = 2 * u
            dqa, dka = issue(a, hidx0, tidx0, q0, k0, sq0, sk0)
            dqb, dkb = issue(a + 1, hidx1, tidx1, q1, k1, sq1, sk1)
            dqa.wait()
            dka.wait()
            compute(a, q0, k0, o0)
            dqb.wait()
            dkb.wait()
            compute(a + 1, q1, k1, o1)
            return 0

        lax.fori_loop(0, n_pairs, pair_body, 0)
        if tail:
            b = n_batches - 1
            dq, dk = issue(b, hidx0, tidx0, q0, k0, sq0, sk0)
            dq.wait()
            dk.wait()
            compute(b, q0, k0, o0)

    buf_set = [
        pltpu.VMEM((ABATCH,), _i32),
        pltpu.VMEM((ABATCH,), _i32),
        pltpu.VMEM((ABATCH, CHANNEL), _f32),
        pltpu.VMEM((ABATCH, CHANNEL), _f32),
        pltpu.VMEM((ABATCH,), _f32),
    ]
    return pl.kernel(
        body,
        out_type=jax.ShapeDtypeStruct((E_pad,), _f32),
        mesh=mesh,
        compiler_params=pltpu.CompilerParams(use_tc_tiling_on_sc=False, needs_layout_passes=False),
        scratch_types=buf_set + buf_set + [pltpu.SemaphoreType.DMA] * 4,
    )


def _make_agg(E_pad, V, R, passes, use_w):
    slice_len = E_pad // NS
    assert slice_len * NS == E_pad
    n_stages = 1
    stage = slice_len
    while stage > 2048:
        n_stages *= 2
        stage = slice_len // n_stages
    assert stage * n_stages == slice_len and stage % L == 0
    cap = stage + BATCH
    rpt = R // NS
    assert rpt * NS == R
    nz, zr = _zero_plan(rpt)
    acc_rows = R + L
    out_rows = NC * passes * R

    mesh = plsc.VectorSubcoreMesh(core_axis_name="c", subcore_axis_name="s")

    def body(src_hbm, head_hbm, tail_hbm, scale_hbm, et_hbm, w_hbm, out_hbm,
             *scr):
        if use_w:
            (h_stage, t_stage, s_stage, e_stage, w_vmem,
             gidx, dstb, sclb, etb, gb_idx, stage2, g_buf, s_buf, acc) = scr
        else:
            (h_stage, t_stage, s_stage,
             gidx, dstb, sclb, gb_idx, stage2, g_buf, s_buf, acc) = scr
            e_stage = etb = w_vmem = None
        c_idx = lax.axis_index("c")
        s_idx = lax.axis_index("s")
        iota = _iota16()
        zeros_f = jnp.zeros((L,), _f32)
        zeros_i = jnp.zeros((L,), _i32)

        if use_w:
            pltpu.sync_copy(w_hbm, w_vmem)

        for p in range(passes):
            chunk = c_idx * passes + p
            lo = chunk * R

            def zrow(r, _):
                for c9 in range(W_COLS // L):
                    s_buf[r, pl.ds(c9 * L, L)] = zeros_f
                return 0
            lax.fori_loop(0, BATCH, zrow, 0)
            row0 = s_idx * rpt
            for i in range(nz):
                pltpu.sync_copy(s_buf.at[pl.ds(0, zr)],
                                acc.at[pl.ds(row0 + i * zr, zr)])
            plsc.subcore_barrier()

            def stage_body(o, _):
                gbase = s_idx * slice_len + o * stage
                pltpu.sync_copy(head_hbm.at[pl.ds(gbase, stage)], h_stage)
                pltpu.sync_copy(tail_hbm.at[pl.ds(gbase, stage)], t_stage)
                pltpu.sync_copy(scale_hbm.at[pl.ds(gbase, stage)], s_stage)
                if use_w:
                    pltpu.sync_copy(et_hbm.at[pl.ds(gbase, stage)], e_stage)

                def blk_body(j, cnt):
                    off = j * L
                    h16 = h_stage[pl.ds(off, L)]
                    m = (h16 >= lo) & (h16 < lo + R)
                    t16 = t_stage[pl.ds(off, L)]
                    s16 = s_stage[pl.ds(off, L)]
                    mi = plsc.cumsum(m.astype(_i32))
                    pos = cnt + mi - 1
                    plsc.store_scatter(gidx, [pos], t16, mask=m)
                    plsc.store_scatter(dstb, [pos], h16 - lo, mask=m)
                    plsc.store_scatter(sclb, [pos], s16, mask=m)
                    if use_w:
                        e16 = e_stage[pl.ds(off, L)]
                        plsc.store_scatter(etb, [pos], e16, mask=m)
                    return cnt + jnp.max(mi)

                cnt = lax.fori_loop(0, stage // L, blk_body, jnp.int32(0))

                for k in range(BATCH // L):
                    base = cnt + k * L
                    gidx[pl.ds(base, L)] = zeros_i
                    dstb[pl.ds(base, L)] = jnp.full((L,), R, _i32)
                    sclb[pl.ds(base, L)] = zeros_f
                    if use_w:
                        etb[pl.ds(base, L)] = zeros_i

                nb = (cnt + (BATCH - 1)) // BATCH

                def batch_body(b, _):
                    boff = b * BATCH
                    for k8 in range(BATCH // L):
                        gb_idx[pl.ds(k8 * L, L)] = gidx[pl.ds(boff + k8 * L, L)]
                        stage2[0, pl.ds(k8 * L, L)] = dstb[pl.ds(boff + k8 * L, L)]
                    pltpu.sync_copy(src_hbm.at[gb_idx], g_buf)

                    ones_f = jnp.ones((L,), _f32)

                    def scale_one(e):
                        eidx = jnp.full((L,), boff + e, _i32)
                        sb = plsc.load_gather(sclb, [eidx])
                        if use_w:
                            et_b = plsc.load_gather(etb, [eidx])
                        for c8 in range(CHANNEL // L):
                            v = g_buf[e, pl.ds(c8 * L, L)]
                            if use_w:
                                wv = plsc.load_gather(
                                    w_vmem, [et_b * CHANNEL + c8 * L + iota])
                                v = v * wv
                            s_buf[e, pl.ds(c8 * L, L)] = v * sb
                        extras = jnp.where(
                            iota == 0, sb,
                            jnp.where(iota == 1, ones_f, zeros_f))
                        s_buf[e, pl.ds(CHANNEL, L)] = extras

                    def ebody(i, _):
                        scale_one(2 * i)
                        scale_one(2 * i + 1)
                        return 0

                    lax.fori_loop(0, BATCH // 2, ebody, 0)
                    pltpu.sync_copy(s_buf, acc.at[stage2.at[0]], add=True)
                    return 0

                lax.fori_loop(0, nb, batch_body, 0)
                return 0

            lax.fori_loop(0, n_stages, stage_body, 0)
            plsc.subcore_barrier()

            out_row0 = chunk * R + row0
            pltpu.sync_copy(acc.at[pl.ds(row0, rpt)],
                            out_hbm.at[pl.ds(out_row0, rpt)])
            plsc.subcore_barrier()

    return pl.kernel(
        body,
        out_type=jax.ShapeDtypeStruct((out_rows, W_COLS), _f32),
        mesh=mesh,
        compiler_params=pltpu.CompilerParams(use_tc_tiling_on_sc=False, needs_layout_passes=False),
        scratch_types=(
            [pltpu.VMEM((stage,), _i32),
             pltpu.VMEM((stage,), _i32),
             pltpu.VMEM((stage,), _f32)]
            + ([pltpu.VMEM((stage,), _i32),
                pltpu.VMEM((10 * CHANNEL,), _f32)] if use_w else [])
            + [pltpu.VMEM((cap,), _i32),
               pltpu.VMEM((cap,), _i32)]
            + [pltpu.VMEM((cap,), _f32)]
            + ([pltpu.VMEM((cap,), _i32)] if use_w else [])
            + [pltpu.VMEM((BATCH,), _i32),
               pltpu.VMEM((1, BATCH), _i32),
               pltpu.VMEM((BATCH, CHANNEL), _f32),
               pltpu.VMEM((BATCH, W_COLS), _f32),
               pltpu.VMEM_SHARED((acc_rows, W_COLS), _f32)]
        ),
    )


_BM = 1000


def _qk_body(x_ref, wq_ref, wk_ref, q_ref, k_ref):
    x = x_ref[...]
    q_ref[...] = jnp.dot(x, wq_ref[...], preferred_element_type=_f32)
    k_ref[...] = jnp.dot(x, wk_ref[...], preferred_element_type=_f32)


def _qk_project(x, wq, wk):
    n = x.shape[0]
    grid = n // _BM
    return pl.pallas_call(
        _qk_body,
        grid=(grid,),
        in_specs=[
            pl.BlockSpec((_BM, CHANNEL), lambda i: (i, 0)),
            pl.BlockSpec((CHANNEL, CHANNEL), lambda i: (0, 0)),
            pl.BlockSpec((CHANNEL, CHANNEL), lambda i: (0, 0)),
        ],
        out_specs=[
            pl.BlockSpec((_BM, CHANNEL), lambda i: (i, 0)),
            pl.BlockSpec((_BM, CHANNEL), lambda i: (i, 0)),
        ],
        out_shape=[
            jax.ShapeDtypeStruct((n, CHANNEL), _f32),
            jax.ShapeDtypeStruct((n, CHANNEL), _f32),
        ],
    )(x, wq, wk)


def _l2n(x):
    n = jnp.sqrt(jnp.sum(x * x, axis=1, keepdims=True))
    return x / jnp.maximum(n, 1e-12)


def _fin_ent_body(raw_ref, res_ref, mean_ref, n_ref, newres_ref):
    raw = raw_ref[...]
    s = raw[:, :CHANNEL]
    c = raw[:, CHANNEL + 1]
    mean = s / jnp.maximum(c, 1.0)[:, None]
    nrm = _l2n(mean)
    mean_ref[...] = mean
    n_ref[...] = nrm
    newres_ref[...] = res_ref[...] + nrm


def _fin_node_body(raw_ref, res_ref, n_ref, newres_ref):
    raw = raw_ref[...]
    s = raw[:, :CHANNEL]
    z = raw[:, CHANNEL]
    c = raw[:, CHANNEL + 1]
    mean = s / (jnp.maximum(z, 1e-30) * jnp.maximum(c, 1.0))[:, None]
    nrm = _l2n(mean)
    n_ref[...] = nrm
    newres_ref[...] = res_ref[...] + nrm


def _fin_user_body(raw_ref, res_ref, newres_ref):
    raw = raw_ref[...]
    nrm = _l2n(raw[:, :CHANNEL])
    newres_ref[...] = res_ref[...] + nrm


def _finalize(body, raw, res, n_out):
    n = res.shape[0]
    grid = n // _BM
    outs = [jax.ShapeDtypeStruct((n, CHANNEL), _f32)] * n_out
    return pl.pallas_call(
        body,
        grid=(grid,),
        in_specs=[
            pl.BlockSpec((_BM, W_COLS), lambda i: (i, 0)),
            pl.BlockSpec((_BM, CHANNEL), lambda i: (i, 0)),
        ],
        out_specs=[pl.BlockSpec((_BM, CHANNEL), lambda i: (i, 0))] * n_out,
        out_shape=outs,
    )(raw, res)


_EPAD_ATTN, _, _, _ = _pad_plan(E_EXTRA, 2048)
_EPAD_KG = _pad_plan(E_KG)[0]
_EPAD_EX = _pad_plan(E_EXTRA)[0]
_NNZ_PAD = _pad_plan(NNZ)[0]

_BISECT = 0

_attn = _make_attn(_EPAD_ATTN)
_agg_ent = _make_agg(_EPAD_KG, N_ENTITIES, 10240, 2, True)
_agg_node = _make_agg(_EPAD_EX, N_NODES, 10240, 3, False)
_agg_user = _make_agg(_NNZ_PAD, N_ENTITIES, 5120, 1, False)


def _pad_i(x, n, val):
    return jnp.concatenate([x, jnp.full((n - x.shape[0],), val, x.dtype)])


def kernel(user_emb, entity_emb, edge_index, edge_type, extra_edge_index,
           extra_edge_type, interact_rows, interact_cols, interact_vals,
           weight, extra_weight, W_Q, W_K):
    del extra_edge_type, extra_weight
    node0 = jnp.concatenate([user_emb, entity_emb], axis=0)

    q, k = _qk_project(node0, W_Q, W_K)
    head_ex = extra_edge_index[0]
    tail_ex = extra_edge_index[1]
    head_ex0 = _pad_i(head_ex, _EPAD_ATTN, 0)
    tail_ex0 = _pad_i(tail_ex, _EPAD_ATTN, 0)
    ee = _attn(q, k, head_ex0, tail_ex0)
    if _BISECT == 1:
        return (entity_emb + ee[0], user_emb, node0)
    if _BISECT == 2:
        head_kg = _pad_i(edge_index[0], _EPAD_KG, BIG)
        tail_kg = _pad_i(edge_index[1], _EPAD_KG, 0)
        et_kg = _pad_i(edge_type - 1, _EPAD_KG, 0)
        ones_kg = jnp.ones((_EPAD_KG,), _f32)
        raw_ent = _agg_ent(entity_emb, head_kg, tail_kg, ones_kg, et_kg,
                           weight.reshape(-1))
        return (entity_emb + raw_ent[0, 0] + ee[0], user_emb, node0)

    head_kg = _pad_i(edge_index[0], _EPAD_KG, BIG)
    tail_kg = _pad_i(edge_index[1], _EPAD_KG, 0)
    et_kg = _pad_i(edge_type - 1, _EPAD_KG, 0)
    ones_kg = jnp.ones((_EPAD_KG,), _f32)
    w_flat = weight.reshape(-1)

    head_exb = _pad_i(head_ex, _EPAD_EX, BIG)
    zeros_i = jnp.zeros((_EPAD_EX,), _i32)
    dummy_w = jnp.zeros((10 * CHANNEL,), _f32)

    rows_p = _pad_i(interact_rows, _NNZ_PAD, BIG)
    cols_p = _pad_i(interact_cols, _NNZ_PAD, 0)
    vals_p = _pad_i(interact_vals, _NNZ_PAD, 0.0)
    zeros_nnz = jnp.zeros((_NNZ_PAD,), _i32)

    ent_res, node_res, user_res = entity_emb, node0, user_emb
    x_ent, x_node = entity_emb, node0
    for _ in range(N_HOPS):
        raw_ent = _agg_ent(x_ent, head_kg, tail_kg, ones_kg, et_kg, w_flat)
        ent_mean, ent_n, ent_res = _finalize(
            _fin_ent_body, raw_ent[:N_ENTITIES], ent_res, 3)

        raw_node = _agg_node(x_node, head_exb, tail_ex0[: _EPAD_EX], ee,
                             zeros_i, dummy_w)
        node_n, node_res = _finalize(
            _fin_node_body, raw_node[:N_NODES], node_res, 2)

        raw_user = _agg_user(ent_mean, rows_p, cols_p, vals_p, zeros_nnz,
                             dummy_w)
        (user_res,) = _finalize(
            _fin_user_body, raw_user[:N_USERS], user_res, 1)

        x_ent, x_node = ent_n, node_n

    return (ent_res, user_res, node_res)
